# baseline (speedup 1.0000x reference)
import jax
import jax.numpy as jnp
from jax.experimental import pallas as pl
from jax.experimental.pallas import tpu as pltpu

SEQ = 2048
DIM = 4096
BLK = 512
NBLK = SEQ // BLK


def _select_body(fetch_ref, m_ref, a_ref, x_ref, o_ref):
    o_ref[...] = jnp.where(m_ref[...] != 0, a_ref[...], x_ref[...])


def kernel(x, attack, attack_mask):
    x2 = x.reshape(SEQ, DIM)
    a2 = attack.reshape(SEQ, DIM)
    m2 = attack_mask.reshape(SEQ, 1).astype(jnp.int32)
    need = jnp.any(m2.reshape(NBLK, BLK) == 0, axis=1)
    idx = jnp.where(need, jnp.arange(NBLK, dtype=jnp.int32), -1)
    fetch = jnp.maximum(jax.lax.cummax(idx), 0)
    out = pl.pallas_call(
        _select_body,
        grid_spec=pltpu.PrefetchScalarGridSpec(
            num_scalar_prefetch=1,
            grid=(NBLK,),
            in_specs=[
                pl.BlockSpec((BLK, 1), lambda i, f: (i, 0)),
                pl.BlockSpec((BLK, DIM), lambda i, f: (i, 0)),
                pl.BlockSpec((BLK, DIM), lambda i, f: (f[i], 0)),
            ],
            out_specs=pl.BlockSpec((BLK, DIM), lambda i, f: (i, 0)),
        ),
        out_shape=jax.ShapeDtypeStruct((SEQ, DIM), x.dtype),
    )(fetch, m2, a2, x2)
    return out.reshape(1, SEQ, DIM)
